# Initial kernel scaffold; baseline (speedup 1.0000x reference)
import jax
import jax.numpy as jnp
from jax.experimental import pallas as pl


def kernel(coords_bb, frames, seq_pos, chain_pos, sample_idx, rbf_ln_g, rbf_ln_b, rbf_W, rbf_b, frame_W, frame_b, seq_emb, edge_ln_g, edge_ln_b, mlp_W0, mlp_b0, mlp_W1, mlp_b1, mlp_W2, mlp_b2):
    raise NotImplementedError("write your pallas kernel here")


# speedup vs baseline: 3.2062x; 3.2062x over previous
import functools

import jax
import jax.numpy as jnp
from jax import lax
from jax.experimental import pallas as pl
from jax.experimental.pallas import tpu as pltpu

_TOP_K = 16
_NUM_RBF = 16
_MIN_RBF, _MAX_RBF = 2.0, 22.0
_SPREAD = (_MAX_RBF - _MIN_RBF) / _NUM_RBF
_RBF_STEP = (_MAX_RBF - _MIN_RBF) / (_NUM_RBF - 1)

_INF = float("inf")
_BIG_I = 2**30

_R = 128
_C = 512
_RB = 64


def _extract_topk(vals, idxs, iota, k):
    out_v, out_i = [], []
    for _ in range(k):
        m = jnp.min(vals, axis=1, keepdims=True)
        pos = jnp.min(jnp.where(vals == m, iota, _BIG_I), axis=1, keepdims=True)
        sel = iota == pos
        out_v.append(m)
        if idxs is None:
            out_i.append(pos)
        else:
            out_i.append(jnp.min(jnp.where(sel, idxs, _BIG_I), axis=1,
                                 keepdims=True))
        vals = jnp.where(sel, _INF, vals)
    return jnp.concatenate(out_v, axis=1), jnp.concatenate(out_i, axis=1)


def _knn_kernel(blk_lo_ref, blk_hi_ref, tile_lo_ref, tile_hi_ref,
                ca_ref, cax_ref, samp_r_ref, sampx_ref,
                nbrs_ref, mask_ref, *, R, C, T, K):
    pid = pl.program_id(0)
    row_lo = blk_lo_ref[pid]
    row_hi = blk_hi_ref[pid]
    car = ca_ref[...]
    xr, yr, zr = car[:, 0:1], car[:, 1:2], car[:, 2:3]
    sr = samp_r_ref[...]
    col_iota = lax.broadcasted_iota(jnp.int32, (R, C), 1)
    pos_iota = lax.broadcasted_iota(jnp.int32, (R, 2 * K), 1)

    def tile_body(t, carry):
        best_d, best_i = carry
        active = jnp.logical_and(tile_lo_ref[t] <= row_hi,
                                 tile_hi_ref[t] >= row_lo)

        def do_tile(bd, bi):
            xc = cax_ref[t, 0:1, :]
            yc = cax_ref[t, 1:2, :]
            zc = cax_ref[t, 2:3, :]
            sc = sampx_ref[t, 0:1, :]
            dx, dy, dz = xc - xr, yc - yr, zc - zr
            d2 = jnp.sqrt(dx * dx + dy * dy + dz * dz)
            d2 = jnp.where(sc != sr, _INF, d2)
            cand_d, cand_p = _extract_topk(d2, None, col_iota, K)
            cand_i = cand_p + t * C
            all_d = jnp.concatenate([bd, cand_d], axis=1)
            all_i = jnp.concatenate([bi, cand_i], axis=1)
            return _extract_topk(all_d, all_i, pos_iota, K)

        return lax.cond(active, do_tile, lambda bd, bi: (bd, bi),
                        best_d, best_i)

    init = (jnp.full((R, K), _INF, jnp.float32),
            jnp.zeros((R, K), jnp.int32))
    best_d, best_i = lax.fori_loop(0, T, tile_body, init)
    valid = best_d < _INF
    row_ids = pid * R + lax.broadcasted_iota(jnp.int32, (R, K), 0)
    nbrs_ref[...] = jnp.where(valid, best_i, row_ids)
    mask_ref[...] = valid.astype(jnp.int32)


def _ln(x, g, b):
    m = jnp.mean(x, axis=1, keepdims=True)
    xm = x - m
    v = jnp.mean(xm * xm, axis=1, keepdims=True)
    return xm / jnp.sqrt(v + 1e-5) * g + b


def _silu(x):
    return x / (1.0 + jnp.exp(-x))


def _edge_kernel(blk_lo_ref, blk_hi_ref, tile_lo_ref, tile_hi_ref,
                 tbl_ref, nbf_ref,
                 rlg_ref, rlb_ref, rbw_ref, rbb_ref,
                 fw_ref, fb_ref, semb_ref,
                 elg_ref, elb_ref,
                 w0_ref, b0_ref, w1_ref, b1_ref, w2_ref, b2_ref,
                 out_ref, *, RB, C, T, K):
    E = RB * K
    pid = pl.program_id(0)
    lo = blk_lo_ref[pid]
    hi = blk_hi_ref[pid]
    nb = nbf_ref[...]
    hp = lax.Precision.HIGHEST
    lane_c = lax.broadcasted_iota(jnp.int32, (E, C), 1)

    def gbody(t, acc):
        active = jnp.logical_and(tile_lo_ref[t] <= hi, tile_hi_ref[t] >= lo)

        def do(a):
            oh = (nb - t * C == lane_c).astype(jnp.float32)
            return a + jnp.dot(oh, tbl_ref[t], precision=hp)

        return lax.cond(active, do, lambda a: a, acc)

    G = lax.fori_loop(0, T, gbody, jnp.zeros((E, 32), jnp.float32))

    rb_per_tile = C // RB
    ce_small = tbl_ref[pid // rb_per_tile,
                       pl.ds((pid % rb_per_tile) * RB, RB), :]
    erow = lax.broadcasted_iota(jnp.int32, (E, 1), 0) // K
    ohc = (erow == lax.broadcasted_iota(jnp.int32, (E, RB), 1)
           ).astype(jnp.float32)
    CE = jnp.dot(ohc, ce_small, precision=hp)

    cent = _MIN_RBF + _RBF_STEP * lax.broadcasted_iota(
        jnp.int32, (E, _NUM_RBF), 1).astype(jnp.float32)
    inv_sp2 = 1.0 / (_SPREAD * _SPREAD)
    parts = []
    for a in range(4):
        for b in range(4):
            dx = CE[:, 3 * a:3 * a + 1] - G[:, 3 * b:3 * b + 1]
            dy = CE[:, 3 * a + 1:3 * a + 2] - G[:, 3 * b + 1:3 * b + 2]
            dz = CE[:, 3 * a + 2:3 * a + 3] - G[:, 3 * b + 2:3 * b + 3]
            dist = jnp.sqrt(dx * dx + dy * dy + dz * dz)
            parts.append(jnp.exp(-((dist - cent) ** 2) * inv_sp2))
    rbf = jnp.concatenate(parts, axis=1)
    rel_rbf = jnp.dot(_ln(rbf, rlg_ref[...], rlb_ref[...]),
                      rbw_ref[...]) + rbb_ref[...]

    acc = None
    for r in range(3):
        for c in range(3):
            col = (CE[:, 12 + r:13 + r] * G[:, 12 + c:13 + c]
                   + CE[:, 15 + r:16 + r] * G[:, 15 + c:16 + c]
                   + CE[:, 18 + r:19 + r] * G[:, 18 + c:19 + c])
            w = fw_ref[3 * r + c:3 * r + c + 1, :]
            acc = col * w if acc is None else acc + col * w
    rel_frames = acc + fb_ref[...]

    sj, si = G[:, 21:22], CE[:, 21:22]
    cj, ci = G[:, 22:23], CE[:, 22:23]
    rel = jnp.clip(sj - si, -32.0, 32.0)
    rel = jnp.where(jnp.round(cj) != jnp.round(ci), 33.0, rel) + 32.0
    ridx = jnp.round(rel).astype(jnp.int32)
    ohs = (ridx == lax.broadcasted_iota(jnp.int32, (E, 128), 1)
           ).astype(jnp.float32)
    rel_emb = jnp.dot(ohs, semb_ref[...], precision=hp)

    h = jnp.concatenate([rel_rbf, rel_frames, rel_emb], axis=1)
    h = _ln(h, elg_ref[...], elb_ref[...])
    h = _silu(jnp.dot(h, w0_ref[...]) + b0_ref[...])
    h = _silu(jnp.dot(h, w1_ref[...]) + b1_ref[...])
    out_ref[...] = jnp.dot(h, w2_ref[...]) + b2_ref[...]


def kernel(coords_bb, frames, seq_pos, chain_pos, sample_idx, rbf_ln_g,
           rbf_ln_b, rbf_W, rbf_b, frame_W, frame_b, seq_emb, edge_ln_g,
           edge_ln_b, mlp_W0, mlp_b0, mlp_W1, mlp_b1, mlp_W2, mlp_b2):
    ZN = coords_bb.shape[0]
    K = _TOP_K
    D = mlp_W1.shape[0]
    R, C, RB = _R, _C, _RB
    NB, T, NB2, E = ZN // R, ZN // C, ZN // RB, RB * K

    Ca = coords_bb[:, 1, :]
    cax = Ca.T.reshape(3, T, C).transpose(1, 0, 2)
    samp = sample_idx.astype(jnp.int32)
    samp_r = samp.reshape(ZN, 1)
    sampx = samp.reshape(1, T, C).transpose(1, 0, 2)
    blk_lo, blk_hi = samp[0::R], samp[R - 1::R]
    tile_lo, tile_hi = samp[0::C], samp[C - 1::C]
    blo2, bhi2 = samp[0::RB], samp[RB - 1::RB]

    grid_a = pltpu.PrefetchScalarGridSpec(
        num_scalar_prefetch=4,
        grid=(NB,),
        in_specs=[
            pl.BlockSpec((R, 3), lambda i, *_: (i, 0)),
            pl.BlockSpec((T, 3, C), lambda i, *_: (0, 0, 0)),
            pl.BlockSpec((R, 1), lambda i, *_: (i, 0)),
            pl.BlockSpec((T, 1, C), lambda i, *_: (0, 0, 0)),
        ],
        out_specs=[
            pl.BlockSpec((R, K), lambda i, *_: (i, 0)),
            pl.BlockSpec((R, K), lambda i, *_: (i, 0)),
        ],
    )
    nbrs, maski = pl.pallas_call(
        functools.partial(_knn_kernel, R=R, C=C, T=T, K=K),
        grid_spec=grid_a,
        out_shape=[jax.ShapeDtypeStruct((ZN, K), jnp.int32),
                   jax.ShapeDtypeStruct((ZN, K), jnp.int32)],
    )(blk_lo, blk_hi, tile_lo, tile_hi, Ca, cax, samp_r, sampx)
    nbr_mask = maski.astype(bool)

    tbl = jnp.concatenate([
        coords_bb.reshape(ZN, 12),
        frames.reshape(ZN, 9),
        seq_pos.astype(jnp.float32).reshape(ZN, 1),
        chain_pos.astype(jnp.float32).reshape(ZN, 1),
        jnp.zeros((ZN, 9), jnp.float32),
    ], axis=1).reshape(T, C, 32)
    nbf = nbrs.reshape(ZN * K, 1)
    semb = jnp.concatenate(
        [seq_emb, jnp.zeros((128 - seq_emb.shape[0], D), jnp.float32)], axis=0)
    row = lambda v: v.reshape(1, -1)

    grid_b = pltpu.PrefetchScalarGridSpec(
        num_scalar_prefetch=4,
        grid=(NB2,),
        in_specs=[
            pl.BlockSpec((T, C, 32), lambda i, *_: (0, 0, 0)),
            pl.BlockSpec((E, 1), lambda i, *_: (i, 0)),
        ] + [pl.BlockSpec(s, lambda i, *_, n=len(s): (0,) * n) for s in [
            (1, 256), (1, 256), (256, D), (1, D),
            (9, D), (1, D), (128, D),
            (1, 3 * D), (1, 3 * D),
            (3 * D, D), (1, D), (D, D), (1, D), (D, D), (1, D),
        ]],
        out_specs=[pl.BlockSpec((E, D), lambda i, *_: (i, 0))],
    )
    edges_flat, = pl.pallas_call(
        functools.partial(_edge_kernel, RB=RB, C=C, T=T, K=K),
        grid_spec=grid_b,
        out_shape=[jax.ShapeDtypeStruct((ZN * K, D), jnp.float32)],
    )(blo2, bhi2, tile_lo, tile_hi, tbl, nbf,
      row(rbf_ln_g), row(rbf_ln_b), rbf_W, row(rbf_b),
      frame_W, row(frame_b), semb,
      row(edge_ln_g), row(edge_ln_b),
      mlp_W0, row(mlp_b0), mlp_W1, row(mlp_b1), mlp_W2, row(mlp_b2))

    edges = edges_flat.reshape(ZN, K, D)
    return edges, nbrs, nbr_mask
